# baseline (speedup 1.0000x reference)
import functools

import jax
import jax.numpy as jnp
from jax import lax
from jax.experimental import pallas as pl
from jax.experimental.pallas import tpu as pltpu
from jax.experimental.pallas import tpu_sc as plsc

_LENGTH = 8192
_CHANNELS = 1024
_SEQ = 8192
_BATCH = 4

_NC = 2
_NS = 16
_NW = _NC * _NS
_B = _SEQ * _BATCH
_BPW = _B // _NW
_C = 32
_G = _BPW // _C
_D = 3

_mesh = plsc.VectorSubcoreMesh(core_axis_name="c", subcore_axis_name="s")


@functools.partial(
    pl.kernel,
    out_type=jax.ShapeDtypeStruct((_B, _CHANNELS), jnp.float32),
    mesh=_mesh,
    scratch_types=[
        pltpu.VMEM((_G, _C), jnp.int32),
        pltpu.VMEM((_D, _C, _CHANNELS), jnp.float32),
        pltpu.SemaphoreType.DMA,
        pltpu.SemaphoreType.DMA,
        pltpu.SemaphoreType.DMA,
        pltpu.SemaphoreType.DMA,
        pltpu.SemaphoreType.DMA,
        pltpu.SemaphoreType.DMA,
    ],
)
def _gather_rows(idx_hbm, table_hbm, out_hbm, idx_v, buf,
                 gs0, gs1, gs2, ss0, ss1, ss2):
    gsems = [gs0, gs1, gs2]
    ssems = [ss0, ss1, ss2]
    wid = lax.axis_index("s") * _NC + lax.axis_index("c")
    base = wid * _BPW
    pltpu.sync_copy(idx_hbm.at[wid], idx_v)

    def start_gather(g):
        b = g % _D
        pltpu.async_copy(table_hbm.at[idx_v.at[g]], buf.at[b], gsems[b])

    def wait_gather(g):
        b = g % _D
        pltpu.make_async_copy(table_hbm.at[idx_v.at[g]], buf.at[b],
                              gsems[b]).wait()

    def start_scatter(g):
        b = g % _D
        pltpu.async_copy(buf.at[b], out_hbm.at[pl.ds(base + g * _C, _C)],
                         ssems[b])

    def wait_scatter(g):
        b = g % _D
        pltpu.make_async_copy(buf.at[b],
                              out_hbm.at[pl.ds(base + g * _C, _C)],
                              ssems[b]).wait()

    start_gather(0)
    start_gather(1)
    for g in range(_G):
        wait_gather(g)
        start_scatter(g)
        if g >= 1:
            wait_scatter(g - 1)
        if g + 2 < _G:
            start_gather(g + 2)
    wait_scatter(_G - 1)


def kernel(i, encoding):
    idx = i.astype(jnp.int32).reshape(_NW, _G, _C)
    out = _gather_rows(idx, encoding)
    return out.reshape(_SEQ, _BATCH, _CHANNELS)
